# baseline (speedup 1.0000x reference)
import functools

import jax
import jax.numpy as jnp
from jax import lax
from jax.experimental import pallas as pl
from jax.experimental.pallas import tpu as pltpu
from jax.experimental.pallas import tpu_sc as plsc

NC = 2
NS = 16
NW = NC * NS

D = 64
C = 128
NBUF = 4

def _gather_kernel(n_chunks: int):
    mesh = plsc.VectorSubcoreMesh(
        core_axis_name="c", subcore_axis_name="s",
        num_cores=NC, num_subcores=NS)
    b_per_w = n_chunks * C

    @functools.partial(
        pl.kernel,
        out_type=jax.ShapeDtypeStruct((NW * b_per_w, 2 * D), jnp.float32),
        name="embed_row_gather",
        mesh=mesh,
        scratch_types=[
            pltpu.VMEM((n_chunks, C), jnp.int32),
            pltpu.VMEM((NBUF, C, D), jnp.float32),
        ] + [pltpu.SemaphoreType.DMA] * NBUF,
        compiler_params=pltpu.CompilerParams(use_tc_tiling_on_sc=False),
    )
    def k(table_hbm, idx_hbm, out_hbm, idx_v, rows_v, *sems):
        wid = lax.axis_index("s") * NC + lax.axis_index("c")
        base = wid * b_per_w
        pltpu.sync_copy(idx_hbm.at[wid], idx_v)

        def start(g, b):
            pltpu.make_async_copy(
                table_hbm.at[idx_v.at[g]], rows_v.at[b], sems[b]).start()

        def finish(g, b):
            pltpu.make_async_copy(
                table_hbm.at[idx_v.at[g]], rows_v.at[b], sems[b]).wait()
            pltpu.sync_copy(
                rows_v.at[b],
                out_hbm.at[pl.ds(base + g * C, C), pl.ds(0, D)])

        for b in range(NBUF):
            start(b, b)

        def body(g0, carry):
            for b in range(NBUF):
                g = g0 + b
                finish(g, b)
                start(g + NBUF, b)
            return carry

        lax.fori_loop(0, (n_chunks - NBUF) // NBUF, lambda i, c: body(i * NBUF, c), 0)
        for b in range(NBUF):
            finish(n_chunks - NBUF + b, b)

    return k


def kernel(x, W_E):
    B, S = x.shape
    n_tok = B * S
    assert n_tok % (NW * C) == 0
    n_chunks = n_tok // (NW * C)
    V = W_E.shape[1]
    table = jnp.pad(W_E.T, ((0, 0), (0, 2 * D - W_E.shape[0]))).reshape(2 * V, D)
    idx = x.reshape(NW, n_chunks, C) * 2
    out = _gather_kernel(n_chunks)(table, idx)
    return out.reshape(B, S, 2, D)[:, :, 0, :]


# speedup vs baseline: 2.8888x; 2.8888x over previous
import functools

import jax
import jax.numpy as jnp
from jax import lax
from jax.experimental import pallas as pl
from jax.experimental.pallas import tpu as pltpu
from jax.experimental.pallas import tpu_sc as plsc

NC = 2
NS = 16
NW = NC * NS

D = 64
C = 128
NBUF = 8

def _gather_kernel(n_chunks: int):
    mesh = plsc.VectorSubcoreMesh(
        core_axis_name="c", subcore_axis_name="s",
        num_cores=NC, num_subcores=NS)
    b_per_w = n_chunks * C

    @functools.partial(
        pl.kernel,
        out_type=jax.ShapeDtypeStruct((NW * b_per_w, 2 * D), jnp.float32),
        name="embed_row_gather",
        mesh=mesh,
        scratch_types=[
            pltpu.VMEM((n_chunks, C), jnp.int32),
            pltpu.VMEM((NBUF, C, D), jnp.float32),
        ] + [pltpu.SemaphoreType.DMA] * NBUF,
        compiler_params=pltpu.CompilerParams(use_tc_tiling_on_sc=False),
    )
    def k(table_hbm, idx_hbm, out_hbm, idx_v, rows_v, *sems):
        wid = lax.axis_index("s") * NC + lax.axis_index("c")
        base = wid * b_per_w
        pltpu.sync_copy(idx_hbm.at[wid], idx_v)

        def start(g, b):
            pltpu.make_async_copy(
                table_hbm.at[idx_v.at[g]], rows_v.at[b], sems[b]).start()

        def finish(g, b):
            pltpu.make_async_copy(
                table_hbm.at[idx_v.at[g]], rows_v.at[b], sems[b]).wait()
            pltpu.sync_copy(
                rows_v.at[b],
                out_hbm.at[pl.ds(base + g * C, C), pl.ds(0, D)])

        for b in range(NBUF):
            start(b, b)

        def body(g0, carry):
            for b in range(NBUF):
                g = g0 + b
                finish(g, b)
                start(g + NBUF, b)
            return carry

        lax.fori_loop(0, (n_chunks - NBUF) // NBUF, lambda i, c: body(i * NBUF, c), 0)
        for b in range(NBUF):
            finish(n_chunks - NBUF + b, b)

    return k


def kernel(x, W_E):
    B, S = x.shape
    n_tok = B * S
    assert n_tok % (NW * C) == 0
    n_chunks = n_tok // (NW * C)
    V = W_E.shape[1]
    table = jnp.pad(W_E.T, ((0, 0), (0, 2 * D - W_E.shape[0]))).reshape(2 * V, D)
    idx = x.reshape(NW, n_chunks, C) * 2
    out = _gather_kernel(n_chunks)(table, idx)
    return out[:, :D].reshape(B, S, D)
